# baseline (speedup 1.0000x reference)
import functools

import numpy as np
import jax
import jax.numpy as jnp
from jax import lax
from jax.experimental import pallas as pl
from jax.experimental.pallas import tpu as pltpu
from jax.experimental.pallas import tpu_sc as plsc

_P = 2038074743
_r = np.random.RandomState(1024)
_rn = np.concatenate([np.array([2038074743]), _r.randint(0, 2038074743, (50,))])
_A, _B = int(_rn[1]), int(_rn[2])
_D = 64
_W = int(1000000 * _D * (1.0 / _D) + 1)
_N = 4096 * 50

_T1 = np.array([(_A * _D * 1024 * h + _B) % _P for h in range(1024)], dtype=np.int32)
_T2 = np.array([(_A * _D * l) % _P for l in range(1024)], dtype=np.int32)
_OFF = [int((_A * d) % _P) for d in range(_D)]
_RECIP = np.float32((1.0 / _W) * (1.0 - 2.0 ** -20))
_PU = np.uint32(_P)

_NC, _NS = 2, 16
_ROWS_PER_SC = _N // _NC
_C = 128
_CD = _C * _D
_CHUNKS_PER_SC = _ROWS_PER_SC // _C
_CHUNKS_PER_TILE = _N // (_NC * _NS) // _C

_mesh = plsc.VectorSubcoreMesh(core_axis_name="c", subcore_axis_name="s")


@functools.partial(
    pl.kernel,
    out_type=jax.ShapeDtypeStruct((_N * _D,), jnp.float32),
    mesh=_mesh,
    compiler_params=pltpu.CompilerParams(needs_layout_passes=False),
    scratch_types=[
        pltpu.VMEM((1024,), jnp.int32),
        pltpu.VMEM((1024,), jnp.int32),
        pltpu.VMEM((_C,), jnp.int32),
        pltpu.VMEM((_C,), jnp.int32),
        pltpu.VMEM((_CD // 2,), jnp.int32),
        pltpu.VMEM((_CD // 2,), jnp.int32),
        pltpu.VMEM((_CD // 2,), jnp.int32),
        pltpu.VMEM((_CD // 2,), jnp.int32),
        pltpu.VMEM((_CD // 2,), jnp.float32),
        pltpu.VMEM((_CD // 2,), jnp.float32),
        pltpu.VMEM((_CD // 2,), jnp.float32),
        pltpu.VMEM((_CD // 2,), jnp.float32),
        pltpu.VMEM_SHARED((_W,), jnp.float32),
        pltpu.SemaphoreType.DMA,
        pltpu.SemaphoreType.DMA,
        pltpu.SemaphoreType.DMA,
        pltpu.SemaphoreType.DMA,
        pltpu.SemaphoreType.DMA,
        pltpu.SemaphoreType.DMA,
        pltpu.SemaphoreType.DMA,
        pltpu.SemaphoreType.DMA,
    ],
)
def _emb_kernel(idx_hbm, w_hbm, t1_hbm, t2_hbm, out_hbm,
                t1_v, t2_v, idx_v, b_v,
                sA0, sB0, sA1, sB1, vA0, vB0, vA1, vB1,
                w_sp, gA0, gB0, gA1, gB1, oA0, oB0, oA1, oB1):
    cid = lax.axis_index("c")
    sid = lax.axis_index("s")
    sc_row0 = jnp.int32(0)
    pltpu.sync_copy(t1_hbm, t1_v)
    pltpu.sync_copy(t2_hbm, t2_v)

    @pl.when(sid == jnp.int32(0))
    def _():
        pltpu.sync_copy(w_hbm, w_sp)

    plsc.subcore_barrier()

    lane64 = lax.iota(jnp.int32, 16) * np.int32(_D)

    def compute_chunk(row0, sa_v, sb_v):
        pltpu.sync_copy(idx_hbm.at[pl.ds(row0, _C)], idx_v)

        def b_body(i, c):
            v = idx_v[pl.ds(i * np.int32(16), 16)]
            hi = lax.shift_right_logical(v, np.int32(10))
            lo = lax.bitwise_and(v, np.int32(1023))
            t1 = plsc.load_gather(t1_v, [hi])
            t2 = plsc.load_gather(t2_v, [lo])
            s = lax.bitcast_convert_type(t1 + t2, jnp.uint32)
            b = jnp.where(s >= _PU, s - _PU, s)
            b_v[pl.ds(i * np.int32(16), 16)] = lax.bitcast_convert_type(b, jnp.int32)
            return c

        lax.fori_loop(jnp.int32(0), jnp.int32(_C // 16), b_body, jnp.int32(0))

        def make_rb_body(slot_v, off):
            def rb_body(rb, c):
                bvec = lax.bitcast_convert_type(
                    b_v[pl.ds(rb * np.int32(16), 16)], jnp.uint32)
                pos0 = lane64 + rb * np.int32(16 * _D) - np.int32(off)
                for d in range(_D):
                    h0 = bvec + np.uint32(_OFF[d])
                    h = jnp.where(h0 >= _PU, h0 - _PU, h0)
                    hi32 = lax.bitcast_convert_type(h, jnp.int32)
                    q = (hi32.astype(jnp.float32) * _RECIP).astype(jnp.int32)
                    r = hi32 - q * np.int32(_W)
                    slot = jnp.where(r >= np.int32(_W), r - np.int32(_W), r)
                    plsc.store_scatter(slot_v, [pos0 + np.int32(d)], slot)
                return c
            return rb_body

        nh = _C // 32
        lax.fori_loop(jnp.int32(0), jnp.int32(nh),
                      make_rb_body(sa_v, 0), jnp.int32(0))
        lax.fori_loop(jnp.int32(nh), jnp.int32(_C // 16),
                      make_rb_body(sb_v, _CD // 2), jnp.int32(0))

    def run_pipeline(nchunk, start_chunk, w_src):
        def chunk_row(k):
            return sc_row0 + (start_chunk + k) * np.int32(_C)

        H = _CD // 2

        def gather_start(bufs):
            for (s_v, v_v, g, o) in bufs:
                pltpu.async_copy(w_src.at[s_v], v_v, g)

        def gather_wait(bufs):
            for (s_v, v_v, g, o) in bufs:
                pltpu.make_async_copy(w_src.at[s_v], v_v, g).wait()

        def out_copy_start(k, bufs):
            o0 = chunk_row(k) * np.int32(_D)
            for i, (s_v, v_v, g, o) in enumerate(bufs):
                pltpu.async_copy(
                    v_v, out_hbm.at[pl.ds(o0 + np.int32(i * H), H)], o)

        def out_copy_wait(k, bufs):
            o0 = chunk_row(k) * np.int32(_D)
            for i, (s_v, v_v, g, o) in enumerate(bufs):
                pltpu.make_async_copy(
                    v_v, out_hbm.at[pl.ds(o0 + np.int32(i * H), H)], o).wait()

        bufs0 = [(sA0, vA0, gA0, oA0), (sB0, vB0, gB0, oB0)]
        bufs1 = [(sA1, vA1, gA1, oA1), (sB1, vB1, gB1, oB1)]

        compute_chunk(chunk_row(jnp.int32(0)), sA0, sB0)
        gather_start(bufs0)

        def sblock(s, carry):
            k1 = np.int32(2) * s + np.int32(1)
            k2 = k1 + np.int32(1)
            compute_chunk(chunk_row(k1), sA1, sB1)
            gather_wait(bufs0)
            out_copy_start(k1 - np.int32(1), bufs0)

            @pl.when(s >= np.int32(1))
            def _():
                out_copy_wait(k1 - np.int32(2), bufs1)

            gather_start(bufs1)
            compute_chunk(chunk_row(k2), sA0, sB0)
            gather_wait(bufs1)
            out_copy_start(k1, bufs1)
            out_copy_wait(k2 - np.int32(2), bufs0)
            gather_start(bufs0)
            return carry

        nsb = (nchunk - 2) // 2
        lax.fori_loop(jnp.int32(0), jnp.int32(nsb), sblock, jnp.int32(0))

        last = np.int32(nchunk - 1)
        compute_chunk(chunk_row(last), sA1, sB1)
        gather_wait(bufs0)
        out_copy_start(last - np.int32(1), bufs0)
        out_copy_wait(last - np.int32(2), bufs1)
        gather_start(bufs1)
        gather_wait(bufs1)
        out_copy_start(last, bufs1)
        out_copy_wait(last - np.int32(1), bufs0)
        out_copy_wait(last, bufs1)

    wid = sid * np.int32(_NC) + cid
    run_pipeline(_CHUNKS_PER_TILE, wid * np.int32(_CHUNKS_PER_TILE), w_sp)


def kernel(indices, hashed_weight):
    idx32 = indices.reshape(-1).astype(jnp.int32)
    w = hashed_weight.astype(jnp.float32)
    out = _emb_kernel(idx32, w, jnp.asarray(_T1), jnp.asarray(_T2))
    return out.reshape(_N, _D)


# speedup vs baseline: 1.0883x; 1.0883x over previous
import functools

import numpy as np
import jax
import jax.numpy as jnp
from jax import lax
from jax.experimental import pallas as pl
from jax.experimental.pallas import tpu as pltpu
from jax.experimental.pallas import tpu_sc as plsc

_P = 2038074743
_r = np.random.RandomState(1024)
_rn = np.concatenate([np.array([2038074743]), _r.randint(0, 2038074743, (50,))])
_A, _B = int(_rn[1]), int(_rn[2])
_D = 64
_W = int(1000000 * _D * (1.0 / _D) + 1)
_N = 4096 * 50

_T1 = np.array([(_A * _D * 1024 * h + _B) % _P for h in range(1024)], dtype=np.int32)
_T2 = np.array([(_A * _D * l) % _P for l in range(1024)], dtype=np.int32)
_OFF = [int((_A * d) % _P) for d in range(_D)]
_RECIP = np.float32((1.0 / _W) * (1.0 - 2.0 ** -20))
_PU = np.uint32(_P)

_NC, _NS = 2, 16
_NW = _NC * _NS
_ROWS_PER_TILE = _N // _NW
_C = 160
_CD = _C * _D
_NCHUNK = _ROWS_PER_TILE // _C

_mesh = plsc.VectorSubcoreMesh(core_axis_name="c", subcore_axis_name="s")


@functools.partial(
    pl.kernel,
    out_type=jax.ShapeDtypeStruct((_N * _D,), jnp.float32),
    mesh=_mesh,
    compiler_params=pltpu.CompilerParams(needs_layout_passes=False),
    scratch_types=[
        pltpu.VMEM((1024,), jnp.int32),
        pltpu.VMEM((1024,), jnp.int32),
        pltpu.VMEM((_C,), jnp.int32),
        pltpu.VMEM((_C,), jnp.int32),
        pltpu.VMEM((_CD,), jnp.int32),
        pltpu.VMEM((_CD,), jnp.int32),
        pltpu.VMEM((_CD,), jnp.float32),
        pltpu.VMEM((_CD,), jnp.float32),
        pltpu.VMEM_SHARED((_W,), jnp.float32),
        pltpu.SemaphoreType.DMA,
        pltpu.SemaphoreType.DMA,
        pltpu.SemaphoreType.DMA,
        pltpu.SemaphoreType.DMA,
    ],
)
def _emb_kernel(idx_hbm, w_hbm, t1_hbm, t2_hbm, out_hbm,
                t1_v, t2_v, idx_v, b_v, slot0_v, slot1_v, val0_v, val1_v,
                w_sp, sem_g0, sem_g1, sem_o0, sem_o1):
    cid = lax.axis_index("c")
    sid = lax.axis_index("s")
    wid = sid * np.int32(_NC) + cid
    pltpu.sync_copy(t1_hbm, t1_v)
    pltpu.sync_copy(t2_hbm, t2_v)

    @pl.when(sid == jnp.int32(0))
    def _():
        pltpu.sync_copy(w_hbm, w_sp)

    plsc.subcore_barrier()

    lane64 = lax.iota(jnp.int32, 16) * np.int32(_D)
    base_row = wid * np.int32(_ROWS_PER_TILE)

    def compute_chunk(row0, slot_v):
        pltpu.sync_copy(idx_hbm.at[pl.ds(row0, _C)], idx_v)

        def b_body(i, c):
            v = idx_v[pl.ds(i * np.int32(16), 16)]
            hi = lax.shift_right_logical(v, np.int32(10))
            lo = lax.bitwise_and(v, np.int32(1023))
            t1 = plsc.load_gather(t1_v, [hi])
            t2 = plsc.load_gather(t2_v, [lo])
            s = lax.bitcast_convert_type(t1 + t2, jnp.uint32)
            b = jnp.where(s >= _PU, s - _PU, s)
            b_v[pl.ds(i * np.int32(16), 16)] = lax.bitcast_convert_type(b, jnp.int32)
            return c

        lax.fori_loop(jnp.int32(0), jnp.int32(_C // 16), b_body, jnp.int32(0))

        def rb_body(rb, c):
            bvec = lax.bitcast_convert_type(
                b_v[pl.ds(rb * np.int32(16), 16)], jnp.uint32)
            pos0 = lane64 + rb * np.int32(16 * _D)
            for d in range(_D):
                h0 = bvec + np.uint32(_OFF[d])
                h = jnp.where(h0 >= _PU, h0 - _PU, h0)
                hi32 = lax.bitcast_convert_type(h, jnp.int32)
                q = (hi32.astype(jnp.float32) * _RECIP).astype(jnp.int32)
                r = hi32 - q * np.int32(_W)
                slot = jnp.where(r >= np.int32(_W), r - np.int32(_W), r)
                plsc.store_scatter(slot_v, [pos0 + np.int32(d)], slot)
            return c

        lax.fori_loop(jnp.int32(0), jnp.int32(_C // 16), rb_body, jnp.int32(0))

    def chunk_row(k):
        return base_row + k * np.int32(_C)

    def gather_start(slot_v, val_v, sem):
        pltpu.async_copy(w_sp.at[slot_v], val_v, sem)

    def gather_wait(slot_v, val_v, sem):
        pltpu.make_async_copy(w_sp.at[slot_v], val_v, sem).wait()

    def out_copy_start(k, val_v, sem):
        pltpu.async_copy(
            val_v, out_hbm.at[pl.ds(chunk_row(k) * np.int32(_D), _CD)], sem)

    def out_copy_wait(k, val_v, sem):
        pltpu.make_async_copy(
            val_v, out_hbm.at[pl.ds(chunk_row(k) * np.int32(_D), _CD)],
            sem).wait()

    compute_chunk(chunk_row(jnp.int32(0)), slot0_v)
    gather_start(slot0_v, val0_v, sem_g0)

    def sblock(s, carry):
        k1 = np.int32(2) * s + np.int32(1)
        k2 = k1 + np.int32(1)
        compute_chunk(chunk_row(k1), slot1_v)
        gather_wait(slot0_v, val0_v, sem_g0)
        out_copy_start(k1 - np.int32(1), val0_v, sem_o0)

        @pl.when(s >= np.int32(1))
        def _():
            out_copy_wait(k1 - np.int32(2), val1_v, sem_o1)

        gather_start(slot1_v, val1_v, sem_g1)
        compute_chunk(chunk_row(k2), slot0_v)
        gather_wait(slot1_v, val1_v, sem_g1)
        out_copy_start(k1, val1_v, sem_o1)
        out_copy_wait(k2 - np.int32(2), val0_v, sem_o0)
        gather_start(slot0_v, val0_v, sem_g0)
        return carry

    nsb = (_NCHUNK - 2) // 2
    lax.fori_loop(jnp.int32(0), jnp.int32(nsb), sblock, jnp.int32(0))

    last = np.int32(_NCHUNK - 1)
    compute_chunk(chunk_row(last), slot1_v)
    gather_wait(slot0_v, val0_v, sem_g0)
    out_copy_start(last - np.int32(1), val0_v, sem_o0)
    out_copy_wait(last - np.int32(2), val1_v, sem_o1)
    gather_start(slot1_v, val1_v, sem_g1)
    gather_wait(slot1_v, val1_v, sem_g1)
    out_copy_start(last, val1_v, sem_o1)
    out_copy_wait(last - np.int32(1), val0_v, sem_o0)
    out_copy_wait(last, val1_v, sem_o1)


def kernel(indices, hashed_weight):
    idx32 = indices.reshape(-1).astype(jnp.int32)
    w = hashed_weight.astype(jnp.float32)
    out = _emb_kernel(idx32, w, jnp.asarray(_T1), jnp.asarray(_T2))
    return out.reshape(_N, _D)


# speedup vs baseline: 1.1316x; 1.0398x over previous
import functools

import numpy as np
import jax
import jax.numpy as jnp
from jax import lax
from jax.experimental import pallas as pl
from jax.experimental.pallas import tpu as pltpu
from jax.experimental.pallas import tpu_sc as plsc

_P = 2038074743
_r = np.random.RandomState(1024)
_rn = np.concatenate([np.array([2038074743]), _r.randint(0, 2038074743, (50,))])
_A, _B = int(_rn[1]), int(_rn[2])
_D = 64
_W = int(1000000 * _D * (1.0 / _D) + 1)
_N = 4096 * 50

_T1 = np.array([(_A * _D * 1024 * h + _B) % _P for h in range(1024)], dtype=np.int32)
_T2 = np.array([(_A * _D * l) % _P for l in range(1024)], dtype=np.int32)
_OFF = [int((_A * d) % _P) for d in range(_D)]
_RECIP = np.float32((1.0 / _W) * (1.0 - 2.0 ** -20))
_PU = np.uint32(_P)

_NC, _NS = 2, 16
_NW = _NC * _NS
_ROWS_PER_TILE = _N // _NW
_C = 160
_CD = _C * _D
_NCHUNK = _ROWS_PER_TILE // _C

_mesh = plsc.VectorSubcoreMesh(core_axis_name="c", subcore_axis_name="s")


@functools.partial(
    pl.kernel,
    out_type=jax.ShapeDtypeStruct((_N * _D,), jnp.float32),
    mesh=_mesh,
    compiler_params=pltpu.CompilerParams(needs_layout_passes=False),
    scratch_types=[
        pltpu.VMEM((1024,), jnp.int32),
        pltpu.VMEM((1024,), jnp.int32),
        pltpu.VMEM((_ROWS_PER_TILE,), jnp.int32),
        pltpu.VMEM((_C,), jnp.int32),
        pltpu.VMEM((_CD,), jnp.int32),
        pltpu.VMEM((_CD,), jnp.int32),
        pltpu.VMEM((_CD,), jnp.float32),
        pltpu.VMEM((_CD,), jnp.float32),
        pltpu.VMEM_SHARED((_W,), jnp.float32),
        pltpu.SemaphoreType.DMA,
        pltpu.SemaphoreType.DMA,
        pltpu.SemaphoreType.DMA,
        pltpu.SemaphoreType.DMA,
    ],
)
def _emb_kernel(idx_hbm, w_hbm, t1_hbm, t2_hbm, out_hbm,
                t1_v, t2_v, idx_v, b_v, slot0_v, slot1_v, val0_v, val1_v,
                w_sp, sem_g0, sem_g1, sem_o0, sem_o1):
    cid = lax.axis_index("c")
    sid = lax.axis_index("s")
    wid = sid * np.int32(_NC) + cid
    pltpu.sync_copy(t1_hbm, t1_v)
    pltpu.sync_copy(t2_hbm, t2_v)

    @pl.when(sid == jnp.int32(0))
    def _():
        pltpu.sync_copy(w_hbm, w_sp)

    plsc.subcore_barrier()

    lane64 = lax.iota(jnp.int32, 16) * np.int32(_D)
    base_row = wid * np.int32(_ROWS_PER_TILE)

    def compute_chunk(off0, slot_v):
        def b_body(i, c):
            v = idx_v[pl.ds(off0 + i * np.int32(16), 16)]
            hi = lax.shift_right_logical(v, np.int32(10))
            lo = lax.bitwise_and(v, np.int32(1023))
            t1 = plsc.load_gather(t1_v, [hi])
            t2 = plsc.load_gather(t2_v, [lo])
            s = lax.bitcast_convert_type(t1 + t2, jnp.uint32)
            b = jnp.where(s >= _PU, s - _PU, s)
            b_v[pl.ds(i * np.int32(16), 16)] = lax.bitcast_convert_type(b, jnp.int32)
            return c

        lax.fori_loop(jnp.int32(0), jnp.int32(_C // 16), b_body, jnp.int32(0))

        def rb_body(rb, c):
            bvec = lax.bitcast_convert_type(
                b_v[pl.ds(rb * np.int32(16), 16)], jnp.uint32)
            pos0 = lane64 + rb * np.int32(16 * _D)
            for d in range(_D):
                h0 = bvec + np.uint32(_OFF[d])
                h = jnp.where(h0 >= _PU, h0 - _PU, h0)
                hi32 = lax.bitcast_convert_type(h, jnp.int32)
                q = (hi32.astype(jnp.float32) * _RECIP).astype(jnp.int32)
                r = hi32 - q * np.int32(_W)
                slot = jnp.where(r >= np.int32(_W), r - np.int32(_W), r)
                plsc.store_scatter(slot_v, [pos0 + np.int32(d)], slot)
            return c

        lax.fori_loop(jnp.int32(0), jnp.int32(_C // 16), rb_body, jnp.int32(0))

    def chunk_row(k):
        return base_row + k * np.int32(_C)

    def gather_start(slot_v, val_v, sem):
        pltpu.async_copy(w_sp.at[slot_v], val_v, sem)

    def gather_wait(slot_v, val_v, sem):
        pltpu.make_async_copy(w_sp.at[slot_v], val_v, sem).wait()

    def out_copy_start(k, val_v, sem):
        pltpu.async_copy(
            val_v, out_hbm.at[pl.ds(chunk_row(k) * np.int32(_D), _CD)], sem)

    def out_copy_wait(k, val_v, sem):
        pltpu.make_async_copy(
            val_v, out_hbm.at[pl.ds(chunk_row(k) * np.int32(_D), _CD)],
            sem).wait()

    pltpu.sync_copy(idx_hbm.at[pl.ds(base_row, _ROWS_PER_TILE)], idx_v)
    compute_chunk(jnp.int32(0), slot0_v)
    gather_start(slot0_v, val0_v, sem_g0)

    def sblock(s, carry):
        k1 = np.int32(2) * s + np.int32(1)
        k2 = k1 + np.int32(1)
        compute_chunk(k1 * np.int32(_C), slot1_v)
        gather_wait(slot0_v, val0_v, sem_g0)
        out_copy_start(k1 - np.int32(1), val0_v, sem_o0)

        @pl.when(s >= np.int32(1))
        def _():
            out_copy_wait(k1 - np.int32(2), val1_v, sem_o1)

        gather_start(slot1_v, val1_v, sem_g1)
        compute_chunk(k2 * np.int32(_C), slot0_v)
        gather_wait(slot1_v, val1_v, sem_g1)
        out_copy_start(k1, val1_v, sem_o1)
        out_copy_wait(k2 - np.int32(2), val0_v, sem_o0)
        gather_start(slot0_v, val0_v, sem_g0)
        return carry

    nsb = (_NCHUNK - 2) // 2
    lax.fori_loop(jnp.int32(0), jnp.int32(nsb), sblock, jnp.int32(0))

    last = np.int32(_NCHUNK - 1)
    compute_chunk(last * np.int32(_C), slot1_v)
    gather_wait(slot0_v, val0_v, sem_g0)
    out_copy_start(last - np.int32(1), val0_v, sem_o0)
    out_copy_wait(last - np.int32(2), val1_v, sem_o1)
    gather_start(slot1_v, val1_v, sem_g1)
    gather_wait(slot1_v, val1_v, sem_g1)
    out_copy_start(last, val1_v, sem_o1)
    out_copy_wait(last - np.int32(1), val0_v, sem_o0)
    out_copy_wait(last, val1_v, sem_o1)


def kernel(indices, hashed_weight):
    idx32 = indices.reshape(-1).astype(jnp.int32)
    w = hashed_weight.astype(jnp.float32)
    out = _emb_kernel(idx32, w, jnp.asarray(_T1), jnp.asarray(_T2))
    return out.reshape(_N, _D)


# speedup vs baseline: 1.1332x; 1.0015x over previous
import functools

import numpy as np
import jax
import jax.numpy as jnp
from jax import lax
from jax.experimental import pallas as pl
from jax.experimental.pallas import tpu as pltpu
from jax.experimental.pallas import tpu_sc as plsc

_P = 2038074743
_r = np.random.RandomState(1024)
_rn = np.concatenate([np.array([2038074743]), _r.randint(0, 2038074743, (50,))])
_A, _B = int(_rn[1]), int(_rn[2])
_D = 64
_W = int(1000000 * _D * (1.0 / _D) + 1)
_N = 4096 * 50

_T1 = np.array([(_A * _D * 1024 * h + _B) % _P for h in range(1024)], dtype=np.int32)
_T2 = np.array([(_A * _D * l) % _P for l in range(1024)], dtype=np.int32)
_OFF = [int((_A * d) % _P) for d in range(_D)]
_RECIP = np.float32((1.0 / _W) * (1.0 - 2.0 ** -20))
_PU = np.uint32(_P)

_NC, _NS = 2, 16
_NW = _NC * _NS
_ROWS_PER_TILE = _N // _NW
_C = 160
_CD = _C * _D
_NCHUNK = _ROWS_PER_TILE // _C

_mesh = plsc.VectorSubcoreMesh(core_axis_name="c", subcore_axis_name="s")


@functools.partial(
    pl.kernel,
    out_type=jax.ShapeDtypeStruct((_N * _D,), jnp.float32),
    mesh=_mesh,
    compiler_params=pltpu.CompilerParams(needs_layout_passes=False, disable_bounds_checks=True, disable_semaphore_checks=True),
    scratch_types=[
        pltpu.VMEM((1024,), jnp.int32),
        pltpu.VMEM((1024,), jnp.int32),
        pltpu.VMEM((_ROWS_PER_TILE,), jnp.int32),
        pltpu.VMEM((_C,), jnp.int32),
        pltpu.VMEM((_CD,), jnp.int32),
        pltpu.VMEM((_CD,), jnp.int32),
        pltpu.VMEM((_CD,), jnp.float32),
        pltpu.VMEM((_CD,), jnp.float32),
        pltpu.VMEM_SHARED((_W,), jnp.float32),
        pltpu.SemaphoreType.DMA,
        pltpu.SemaphoreType.DMA,
        pltpu.SemaphoreType.DMA,
        pltpu.SemaphoreType.DMA,
    ],
)
def _emb_kernel(idx_hbm, w_hbm, t1_hbm, t2_hbm, out_hbm,
                t1_v, t2_v, idx_v, b_v, slot0_v, slot1_v, val0_v, val1_v,
                w_sp, sem_g0, sem_g1, sem_o0, sem_o1):
    cid = lax.axis_index("c")
    sid = lax.axis_index("s")
    wid = sid * np.int32(_NC) + cid
    pltpu.sync_copy(t1_hbm, t1_v)
    pltpu.sync_copy(t2_hbm, t2_v)

    @pl.when(sid == jnp.int32(0))
    def _():
        pltpu.sync_copy(w_hbm, w_sp)

    plsc.subcore_barrier()

    lane64 = lax.iota(jnp.int32, 16) * np.int32(_D)
    base_row = wid * np.int32(_ROWS_PER_TILE)

    def compute_chunk(off0, slot_v):
        def b_body(i, c):
            v = idx_v[pl.ds(off0 + i * np.int32(16), 16)]
            hi = lax.shift_right_logical(v, np.int32(10))
            lo = lax.bitwise_and(v, np.int32(1023))
            t1 = plsc.load_gather(t1_v, [hi])
            t2 = plsc.load_gather(t2_v, [lo])
            s = lax.bitcast_convert_type(t1 + t2, jnp.uint32)
            b = jnp.where(s >= _PU, s - _PU, s)
            b_v[pl.ds(i * np.int32(16), 16)] = lax.bitcast_convert_type(b, jnp.int32)
            return c

        lax.fori_loop(jnp.int32(0), jnp.int32(_C // 16), b_body, jnp.int32(0))

        def rb_body(rb, c):
            bvec = lax.bitcast_convert_type(
                b_v[pl.ds(rb * np.int32(16), 16)], jnp.uint32)
            pos0 = lane64 + rb * np.int32(16 * _D)
            for d in range(_D):
                h0 = bvec + np.uint32(_OFF[d])
                h = jnp.where(h0 >= _PU, h0 - _PU, h0)
                hi32 = lax.bitcast_convert_type(h, jnp.int32)
                q = (hi32.astype(jnp.float32) * _RECIP).astype(jnp.int32)
                r = hi32 - q * np.int32(_W)
                slot = jnp.where(r >= np.int32(_W), r - np.int32(_W), r)
                plsc.store_scatter(slot_v, [pos0 + np.int32(d)], slot)
            return c

        lax.fori_loop(jnp.int32(0), jnp.int32(_C // 16), rb_body, jnp.int32(0))

    def chunk_row(k):
        return base_row + k * np.int32(_C)

    def gather_start(slot_v, val_v, sem):
        pltpu.async_copy(w_sp.at[slot_v], val_v, sem)

    def gather_wait(slot_v, val_v, sem):
        pltpu.make_async_copy(w_sp.at[slot_v], val_v, sem).wait()

    def out_copy_start(k, val_v, sem):
        pltpu.async_copy(
            val_v, out_hbm.at[pl.ds(chunk_row(k) * np.int32(_D), _CD)], sem)

    def out_copy_wait(k, val_v, sem):
        pltpu.make_async_copy(
            val_v, out_hbm.at[pl.ds(chunk_row(k) * np.int32(_D), _CD)],
            sem).wait()

    pltpu.sync_copy(idx_hbm.at[pl.ds(base_row, _ROWS_PER_TILE)], idx_v)
    compute_chunk(jnp.int32(0), slot0_v)
    gather_start(slot0_v, val0_v, sem_g0)

    def sblock(s, carry):
        k1 = np.int32(2) * s + np.int32(1)
        k2 = k1 + np.int32(1)
        compute_chunk(k1 * np.int32(_C), slot1_v)
        gather_wait(slot0_v, val0_v, sem_g0)
        out_copy_start(k1 - np.int32(1), val0_v, sem_o0)

        @pl.when(s >= np.int32(1))
        def _():
            out_copy_wait(k1 - np.int32(2), val1_v, sem_o1)

        gather_start(slot1_v, val1_v, sem_g1)
        compute_chunk(k2 * np.int32(_C), slot0_v)
        gather_wait(slot1_v, val1_v, sem_g1)
        out_copy_start(k1, val1_v, sem_o1)
        out_copy_wait(k2 - np.int32(2), val0_v, sem_o0)
        gather_start(slot0_v, val0_v, sem_g0)
        return carry

    nsb = (_NCHUNK - 2) // 2
    lax.fori_loop(jnp.int32(0), jnp.int32(nsb), sblock, jnp.int32(0))

    last = np.int32(_NCHUNK - 1)
    compute_chunk(last * np.int32(_C), slot1_v)
    gather_wait(slot0_v, val0_v, sem_g0)
    out_copy_start(last - np.int32(1), val0_v, sem_o0)
    out_copy_wait(last - np.int32(2), val1_v, sem_o1)
    gather_start(slot1_v, val1_v, sem_g1)
    gather_wait(slot1_v, val1_v, sem_g1)
    out_copy_start(last, val1_v, sem_o1)
    out_copy_wait(last - np.int32(1), val0_v, sem_o0)
    out_copy_wait(last, val1_v, sem_o1)


def kernel(indices, hashed_weight):
    idx32 = indices.reshape(-1).astype(jnp.int32)
    w = hashed_weight.astype(jnp.float32)
    out = _emb_kernel(idx32, w, jnp.asarray(_T1), jnp.asarray(_T2))
    return out.reshape(_N, _D)
